# Initial kernel scaffold; baseline (speedup 1.0000x reference)
import jax
import jax.numpy as jnp
from jax.experimental import pallas as pl


def kernel(node_feats, edge_index, graph_ids, W0, a_l0, a_r0, W1, a_l1, a_r1, g0_w, g0_b, g0_a, g1_w, g1_b, g1_a):
    raise NotImplementedError("write your pallas kernel here")


# speedup vs baseline: 14.4393x; 14.4393x over previous
import functools

import jax
import jax.numpy as jnp
from jax import lax
from jax.experimental import pallas as pl
from jax.experimental.pallas import tpu as pltpu
from jax.experimental.pallas import tpu_sc as plsc

N = 10000
E = 320000
B = 100
D = 128
H = 2
R = 144
L = 16
NC = 2
NS = 16
BN = 1000
GRID = N // BN
EW = E // NS
HE = E // NC
CS = 800
QCH = 80
CAPQ = 12160
QMAX = CAPQ - QCH
RT = N // NS
RTP = 640


def _t0_body(x_ref, w_ref, t_ref):
    h = jnp.dot(x_ref[...], w_ref[...], preferred_element_type=jnp.float32)
    ones = jnp.ones((BN, 1), jnp.float32)
    zeros = jnp.zeros((BN, R - D - 1), jnp.float32)
    for hd in range(H):
        t_ref[hd] = jnp.concatenate(
            [h[:, hd * D:(hd + 1) * D], ones, zeros], axis=1)


_t0_call = pl.pallas_call(
    _t0_body,
    grid=(GRID,),
    in_specs=[pl.BlockSpec((BN, D), lambda i: (i, 0)),
              pl.BlockSpec((D, H * D), lambda i: (0, 0))],
    out_specs=pl.BlockSpec((H, BN, R), lambda i: (0, i, 0)),
    out_shape=jax.ShapeDtypeStruct((H, N, R), jnp.float32),
)


def _eler_body(x_ref, w_ref, al_ref, ar_ref, o_ref):
    w = w_ref[...]
    f2 = al_ref.shape[1]
    cols = []
    for a_ref in (al_ref, ar_ref):
        a = a_ref[...]
        vh = []
        for hd in range(H):
            wh = w[:, hd * f2:(hd + 1) * f2]
            vh.append(lax.dot_general(
                wh, a[hd:hd + 1, :], (((1,), (1,)), ((), ())),
                preferred_element_type=jnp.float32))
        cols.extend(vh)
    v = jnp.concatenate([cols[0], cols[1], cols[2], cols[3]], axis=1)
    o_ref[...] = lax.dot_general(
        v, x_ref[...], (((0,), (1,)), ((), ())),
        preferred_element_type=jnp.float32)


def _make_eler(f, f2):
    return pl.pallas_call(
        _eler_body,
        in_specs=[pl.BlockSpec((N, f), lambda: (0, 0)),
                  pl.BlockSpec((f, H * f2), lambda: (0, 0)),
                  pl.BlockSpec((H, f2), lambda: (0, 0)),
                  pl.BlockSpec((H, f2), lambda: (0, 0))],
        out_specs=pl.BlockSpec((4, N), lambda: (0, 0)),
        out_shape=jax.ShapeDtypeStruct((4, N), jnp.float32),
    )


_eler0_call = _make_eler(D, D)
_eler1_call = _make_eler(H * D, H * D)


def _stats_body(nt, o_ref, g_ref, s1_ref, s2_ref, cnt_ref):
    i = pl.program_id(0)
    g = g_ref[0, 0, :]
    m = (lax.broadcasted_iota(jnp.int32, (B, BN), 0)
         == g[None, :]).astype(jnp.float32)

    @pl.when(i == 0)
    def _():
        s1_ref[...] = jnp.zeros_like(s1_ref)
        s2_ref[...] = jnp.zeros_like(s2_ref)
        cnt_ref[...] = jnp.zeros_like(cnt_ref)

    cnt_ref[...] += jnp.broadcast_to(
        jnp.sum(m, axis=1, keepdims=True), (B, D))
    for t in range(nt):
        ot = o_ref[t]
        agg = ot[:, :D] / (ot[:, D:D + 1] + 1e-16)
        s1_ref[:, t * D:(t + 1) * D] += jnp.dot(
            m, agg, preferred_element_type=jnp.float32)
        s2_ref[:, t * D:(t + 1) * D] += jnp.dot(
            m, agg * agg, preferred_element_type=jnp.float32)


def _make_stats(nt):
    return pl.pallas_call(
        functools.partial(_stats_body, nt),
        grid=(GRID,),
        in_specs=[pl.BlockSpec((nt, BN, R), lambda i: (0, i, 0)),
                  pl.BlockSpec((1, 1, BN), lambda i: (i, 0, 0))],
        out_specs=[pl.BlockSpec((B, nt * D), lambda i: (0, 0)),
                   pl.BlockSpec((B, nt * D), lambda i: (0, 0)),
                   pl.BlockSpec((B, D), lambda i: (0, 0))],
        out_shape=[jax.ShapeDtypeStruct((B, nt * D), jnp.float32),
                   jax.ShapeDtypeStruct((B, nt * D), jnp.float32),
                   jax.ShapeDtypeStruct((B, D), jnp.float32)],
    )


_stats2_call = _make_stats(H)
_stats4_call = _make_stats(2 * H)


def _norm_parts(nt, o_ref, g, s1_ref, s2_ref, cnt, w_ref, b_ref, ms_ref):
    mt = (g[:, None] == lax.broadcasted_iota(
        jnp.int32, (BN, B), 1)).astype(jnp.float32)
    mean = s1_ref[...] / cnt
    ms = ms_ref[...]
    var = s2_ref[...] / cnt - mean * mean * ms * (2.0 - ms)
    mean_n = jnp.dot(mt, mean, preferred_element_type=jnp.float32)
    var_n = jnp.dot(mt, var, preferred_element_type=jnp.float32)
    ys = []
    for t in range(nt):
        ot = o_ref[t]
        agg = ot[:, :D] / (ot[:, D:D + 1] + 1e-16)
        sl = slice(t * D, (t + 1) * D)
        yt = (w_ref[sl] * (agg - ms[sl] * mean_n[:, sl])
              * lax.rsqrt(var_n[:, sl] + 1e-5) + b_ref[sl])
        ys.append(jnp.where(yt >= 0.0, yt, 0.01 * yt))
    return ys


def _post0_body(o_ref, g_ref, s1_ref, s2_ref, cnt_ref, w_ref, b_ref, ms_ref,
                w1_ref, y_ref, t1_ref, r0_ref, racc):
    i = pl.program_id(0)
    g = g_ref[0, 0, :]
    cnt = jnp.maximum(cnt_ref[:, 0:1], 1.0)
    ys = _norm_parts(H, o_ref, g, s1_ref, s2_ref, cnt, w_ref, b_ref, ms_ref)
    y = jnp.concatenate(ys, axis=1)
    y_ref[...] = y
    m = (lax.broadcasted_iota(jnp.int32, (B, BN), 0)
         == g[None, :]).astype(jnp.float32)

    @pl.when(i == 0)
    def _():
        racc[...] = jnp.zeros_like(racc)

    racc[...] += jnp.dot(m, 0.5 * (ys[0] + ys[1]),
                         preferred_element_type=jnp.float32)

    @pl.when(i == GRID - 1)
    def _():
        r0_ref[...] = racc[...] / cnt

    h1 = jnp.dot(y, w1_ref[...], preferred_element_type=jnp.float32)
    ones = jnp.ones((BN, 1), jnp.float32)
    zeros = jnp.zeros((BN, R - D - 1), jnp.float32)
    for t in range(2 * H):
        hd, q = t // 2, t % 2
        base = hd * (H * D) + q * D
        t1_ref[t] = jnp.concatenate(
            [h1[:, base:base + D], ones, zeros], axis=1)


_post0_call = pl.pallas_call(
    _post0_body,
    grid=(GRID,),
    in_specs=[pl.BlockSpec((H, BN, R), lambda i: (0, i, 0)),
              pl.BlockSpec((1, 1, BN), lambda i: (i, 0, 0)),
              pl.BlockSpec((B, H * D), lambda i: (0, 0)),
              pl.BlockSpec((B, H * D), lambda i: (0, 0)),
              pl.BlockSpec((B, D), lambda i: (0, 0)),
              pl.BlockSpec((H * D,), lambda i: (0,)),
              pl.BlockSpec((H * D,), lambda i: (0,)),
              pl.BlockSpec((H * D,), lambda i: (0,)),
              pl.BlockSpec((H * D, 2 * H * D), lambda i: (0, 0))],
    out_specs=[pl.BlockSpec((BN, H * D), lambda i: (i, 0)),
               pl.BlockSpec((2 * H, BN, R), lambda i: (0, i, 0)),
               pl.BlockSpec((B, D), lambda i: (0, 0))],
    out_shape=[jax.ShapeDtypeStruct((N, H * D), jnp.float32),
               jax.ShapeDtypeStruct((2 * H, N, R), jnp.float32),
               jax.ShapeDtypeStruct((B, D), jnp.float32)],
    scratch_shapes=[pltpu.VMEM((B, D), jnp.float32)],
)


def _final_body(o_ref, g_ref, s1_ref, s2_ref, cnt_ref, w_ref, b_ref, ms_ref,
                r0_ref, out_ref, racc):
    i = pl.program_id(0)
    g = g_ref[0, 0, :]
    cnt = jnp.maximum(cnt_ref[:, 0:1], 1.0)
    ys = _norm_parts(2 * H, o_ref, g, s1_ref, s2_ref, cnt,
                     w_ref, b_ref, ms_ref)
    feats1 = 0.5 * (jnp.concatenate([ys[0], ys[1]], axis=1)
                    + jnp.concatenate([ys[2], ys[3]], axis=1))
    m = (lax.broadcasted_iota(jnp.int32, (B, BN), 0)
         == g[None, :]).astype(jnp.float32)

    @pl.when(i == 0)
    def _():
        racc[...] = jnp.zeros_like(racc)

    racc[...] += jnp.dot(m, feats1, preferred_element_type=jnp.float32)

    @pl.when(i == GRID - 1)
    def _():
        r0 = r0_ref[...]
        r1 = racc[...] / cnt
        out_ref[:, :D] = jnp.where(r0 >= 0.0, r0, 0.01 * r0)
        out_ref[:, D:] = jnp.where(r1 >= 0.0, r1, 0.01 * r1)


_final_call = pl.pallas_call(
    _final_body,
    grid=(GRID,),
    in_specs=[pl.BlockSpec((2 * H, BN, R), lambda i: (0, i, 0)),
              pl.BlockSpec((1, 1, BN), lambda i: (i, 0, 0)),
              pl.BlockSpec((B, 2 * H * D), lambda i: (0, 0)),
              pl.BlockSpec((B, 2 * H * D), lambda i: (0, 0)),
              pl.BlockSpec((B, D), lambda i: (0, 0)),
              pl.BlockSpec((2 * H * D,), lambda i: (0,)),
              pl.BlockSpec((2 * H * D,), lambda i: (0,)),
              pl.BlockSpec((2 * H * D,), lambda i: (0,)),
              pl.BlockSpec((B, D), lambda i: (0, 0))],
    out_specs=pl.BlockSpec((B, 3 * D), lambda i: (0, 0)),
    out_shape=jax.ShapeDtypeStruct((B, 3 * D), jnp.float32),
    scratch_shapes=[pltpu.VMEM((B, H * D), jnp.float32)],
)


_SC_PARAMS = dict(
    compiler_params=pltpu.CompilerParams(needs_layout_passes=False,
                                         use_tc_tiling_on_sc=False),
)


def _sc_mesh():
    return plsc.VectorSubcoreMesh(core_axis_name="c", subcore_axis_name="s",
                                  num_cores=NC, num_subcores=NS)


def _iota16():
    return lax.broadcasted_iota(jnp.int32, (L,), 0)


def _bucket_body(src_hbm, dst_hbm, qsrc_hbm, qdst_hbm, qcnt_hbm,
                 srcb, dstb, qsrc_v, qdst_v, qcv):
    c = lax.axis_index("c")
    s = lax.axis_index("s")
    lo = s * RT
    hi = lo + RT

    def _chunk(ch, qp):
        off = c * HE + ch * CS
        pltpu.sync_copy(src_hbm.at[pl.ds(off, CS)], srcb)
        pltpu.sync_copy(dst_hbm.at[pl.ds(off, CS)], dstb)

        def _scan(i, qp2):
            sv = srcb[pl.ds(i * L, L)]
            dv = dstb[pl.ds(i * L, L)]
            m = (dv >= lo) & (dv < hi)
            plsc.store_compressed(qsrc_v.at[pl.ds(qp2, L)], sv, mask=m)
            plsc.store_compressed(qdst_v.at[pl.ds(qp2, L)], dv - lo, mask=m)
            qn = plsc.all_reduce_population_count(m)
            return jnp.minimum(qp2 + jnp.max(qn), QMAX)
        return lax.fori_loop(0, CS // L, _scan, qp)
    qp = lax.fori_loop(0, HE // CS, _chunk, jnp.int32(0))

    padsrc = jnp.full((L,), N, jnp.int32)
    padzero = jnp.zeros((L,), jnp.int32)
    for k in range(QCH // L):
        qsrc_v[pl.ds(qp + k * L, L)] = padsrc
        qdst_v[pl.ds(qp + k * L, L)] = padzero
    nch = (qp + (QCH - 1)) // QCH
    qcv[...] = jnp.full((L,), nch, jnp.int32)
    pltpu.sync_copy(qsrc_v, qsrc_hbm.at[c, s])
    pltpu.sync_copy(qdst_v, qdst_hbm.at[c, s])
    pltpu.sync_copy(qcv, qcnt_hbm.at[c, s])


def _make_bucket():
    return functools.partial(
        pl.kernel,
        out_type=(jax.ShapeDtypeStruct((NC, NS, CAPQ), jnp.int32),
                  jax.ShapeDtypeStruct((NC, NS, CAPQ), jnp.int32),
                  jax.ShapeDtypeStruct((NC, NS, L), jnp.int32)),
        mesh=_sc_mesh(),
        scratch_types=[
            pltpu.VMEM((CS,), jnp.int32),
            pltpu.VMEM((CS,), jnp.int32),
            pltpu.VMEM((CAPQ,), jnp.int32),
            pltpu.VMEM((CAPQ,), jnp.int32),
            pltpu.VMEM((L,), jnp.int32),
        ],
        **_SC_PARAMS,
    )(_bucket_body)


def _attn_body(qsrc_hbm, qdst_hbm, qcnt_hbm, eler_hbm, ex_hbm,
               el_v, er_v, srcb, dstb, exb, qcv):
    c = lax.axis_index("c")
    s = lax.axis_index("s")
    lo = s * RT
    pltpu.sync_copy(eler_hbm.at[c], el_v)
    pltpu.sync_copy(eler_hbm.at[c + 2, pl.ds(0, N)], er_v)
    for half in range(NC):
        pltpu.sync_copy(qcnt_hbm.at[half, s], qcv)
        nch = jnp.max(qcv[...])

        def _chunk(ch, carry):
            off = ch * QCH
            pltpu.sync_copy(qsrc_hbm.at[half, s, pl.ds(off, QCH)], srcb)
            pltpu.sync_copy(qdst_hbm.at[half, s, pl.ds(off, QCH)], dstb)

            def _att(i, carry2):
                sv = srcb[pl.ds(i * L, L)]
                dv = dstb[pl.ds(i * L, L)] + lo
                e = plsc.load_gather(el_v, [sv]) + plsc.load_gather(er_v, [dv])
                e = jnp.where(e >= 0.0, e, 0.2 * e)
                exb[pl.ds(i * L, L)] = jnp.exp(e)
                return carry2
            lax.fori_loop(0, QCH // L, _att, 0)
            pltpu.sync_copy(exb, ex_hbm.at[c, half, s, pl.ds(off, QCH)])
            return carry
        lax.fori_loop(0, nch, _chunk, 0)


def _make_attn():
    return functools.partial(
        pl.kernel,
        out_type=jax.ShapeDtypeStruct((NC, NC, NS, CAPQ), jnp.float32),
        mesh=_sc_mesh(),
        scratch_types=[
            pltpu.VMEM((N + L,), jnp.float32),
            pltpu.VMEM((N,), jnp.float32),
            pltpu.VMEM((QCH,), jnp.int32),
            pltpu.VMEM((QCH,), jnp.int32),
            pltpu.VMEM((QCH,), jnp.float32),
            pltpu.VMEM((L,), jnp.int32),
        ],
        **_SC_PARAMS,
    )(_attn_body)


def _edge_body(n_passes, t_hbm, qsrc_hbm, qdst_hbm, qcnt_hbm, ex_hbm, o_hbm,
               srcb, dstb, exb, rows, acc, qcv, sem):
    c = lax.axis_index("c")
    s = lax.axis_index("s")
    n_tab = NC * n_passes
    zero16 = jnp.zeros((L,), jnp.float32)
    lanes = [_iota16() + j * L for j in range(R // L)]

    for q in range(n_passes):
        t = c * n_passes + q
        t_off = t * N

        def _zrow(k, carry):
            for j in range(R // L):
                acc[k, pl.ds(j * L, L)] = zero16
            return carry
        lax.fori_loop(0, RTP, _zrow, 0)

        for half in range(NC):
            pltpu.sync_copy(qcnt_hbm.at[half, s], qcv)
            nch = jnp.max(qcv[...])

            def _chunk(ch, carry):
                off = ch * QCH
                pltpu.sync_copy(qsrc_hbm.at[half, s, pl.ds(off, QCH)], srcb)
                pltpu.sync_copy(qdst_hbm.at[half, s, pl.ds(off, QCH)], dstb)
                pltpu.sync_copy(ex_hbm.at[c, half, s, pl.ds(off, QCH)], exb)

                def _boff(i, carry2):
                    sv = srcb[pl.ds(i * L, L)]
                    srcb[pl.ds(i * L, L)] = jnp.minimum(
                        sv + t_off, n_tab * N - 1)
                    return carry2
                lax.fori_loop(0, QCH // L, _boff, 0)
                pltpu.async_copy(t_hbm.at[srcb], rows, sem).wait()

                def _accum(i, carry2):
                    isp = jnp.full((L,), i, jnp.int32)
                    li = plsc.load_gather(dstb, [isp])
                    a = plsc.load_gather(exb, [isp])
                    for j in range(R // L):
                        r = rows[i, pl.ds(j * L, L)] * a
                        plsc.addupdate_scatter(acc, [li, lanes[j]], r)
                    return carry2
                lax.fori_loop(0, QCH, _accum, 0)
                return carry
            lax.fori_loop(0, nch, _chunk, 0)

        pltpu.sync_copy(acc.at[pl.ds(0, RT)],
                        o_hbm.at[pl.ds(t_off + s * RT, RT)])


def _make_edge(n_passes):
    n_tab = NC * n_passes
    return functools.partial(
        pl.kernel,
        out_type=jax.ShapeDtypeStruct((n_tab * N, R), jnp.float32),
        mesh=_sc_mesh(),
        scratch_types=[
            pltpu.VMEM((QCH,), jnp.int32),
            pltpu.VMEM((QCH,), jnp.int32),
            pltpu.VMEM((QCH,), jnp.float32),
            pltpu.VMEM((QCH, R), jnp.float32),
            pltpu.VMEM((RTP, R), jnp.float32),
            pltpu.VMEM((L,), jnp.int32),
            pltpu.SemaphoreType.DMA,
        ],
        **_SC_PARAMS,
    )(functools.partial(_edge_body, n_passes))


_make_bucket = functools.lru_cache(maxsize=None)(_make_bucket)
_make_attn = functools.lru_cache(maxsize=None)(_make_attn)
_make_edge = functools.lru_cache(maxsize=None)(_make_edge)


def kernel(node_feats, edge_index, graph_ids, W0, a_l0, a_r0, W1, a_l1, a_r1,
           g0_w, g0_b, g0_a, g1_w, g1_b, g1_a):
    src = edge_index[0]
    dst = edge_index[1]
    gids3 = graph_ids.reshape(GRID, 1, BN)

    qsrc, qdst, qcnt = _make_bucket()(src, dst)

    t0 = _t0_call(node_feats, W0)
    eler0 = _eler0_call(node_feats, W0, a_l0, a_r0)
    eler0p = jnp.pad(eler0, ((0, 0), (0, L)), constant_values=-1e30)
    ex0 = _make_attn()(qsrc, qdst, qcnt, eler0p)
    o0 = _make_edge(1)(t0.reshape(H * N, R), qsrc, qdst, qcnt, ex0)
    o0 = o0.reshape(H, N, R)

    w0t = jnp.concatenate([g0_w, g0_w])
    b0t = jnp.concatenate([g0_b, g0_b])
    ms0t = jnp.concatenate([g0_a, g0_a])
    s1, s2, cnt = _stats2_call(o0, gids3)
    y, t1, r0 = _post0_call(o0, gids3, s1, s2, cnt, w0t, b0t, ms0t, W1)

    eler1 = _eler1_call(y, W1, a_l1, a_r1)
    eler1p = jnp.pad(eler1, ((0, 0), (0, L)), constant_values=-1e30)
    ex1 = _make_attn()(qsrc, qdst, qcnt, eler1p)
    o1 = _make_edge(2)(t1.reshape(2 * H * N, R), qsrc, qdst, qcnt, ex1)
    o1 = o1.reshape(2 * H, N, R)

    w1t = jnp.concatenate([g1_w, g1_w])
    b1t = jnp.concatenate([g1_b, g1_b])
    ms1t = jnp.concatenate([g1_a, g1_a])
    s1b, s2b, _ = _stats4_call(o1, gids3)
    return _final_call(o1, gids3, s1b, s2b, cnt, w1t, b1t, ms1t, r0)


# speedup vs baseline: 19.5014x; 1.3506x over previous
import functools

import jax
import jax.numpy as jnp
from jax import lax
from jax.experimental import pallas as pl
from jax.experimental.pallas import tpu as pltpu
from jax.experimental.pallas import tpu_sc as plsc

N = 10000
E = 320000
B = 100
D = 128
H = 2
R = 144
L = 16
NC = 2
NS = 16
BN = 1000
GRID = N // BN
EW = E // NS
HE = E // NC
CS = 1600
QCH = 80
CA = 1040
CAPQ = 13520
QMAX = 12080
RT = N // NS
RTP = 640


def _t0_body(x_ref, w_ref, t_ref):
    h = jnp.dot(x_ref[...], w_ref[...], preferred_element_type=jnp.float32)
    ones = jnp.ones((BN, 1), jnp.float32)
    zeros = jnp.zeros((BN, R - D - 1), jnp.float32)
    for hd in range(H):
        t_ref[hd] = jnp.concatenate(
            [h[:, hd * D:(hd + 1) * D], ones, zeros], axis=1)


_t0_call = pl.pallas_call(
    _t0_body,
    grid=(GRID,),
    in_specs=[pl.BlockSpec((BN, D), lambda i: (i, 0)),
              pl.BlockSpec((D, H * D), lambda i: (0, 0))],
    out_specs=pl.BlockSpec((H, BN, R), lambda i: (0, i, 0)),
    out_shape=jax.ShapeDtypeStruct((H, N, R), jnp.float32),
)


def _eler_body(x_ref, w_ref, al_ref, ar_ref, o_ref):
    w = w_ref[...]
    f2 = al_ref.shape[1]
    cols = []
    for a_ref in (al_ref, ar_ref):
        a = a_ref[...]
        vh = []
        for hd in range(H):
            wh = w[:, hd * f2:(hd + 1) * f2]
            vh.append(lax.dot_general(
                wh, a[hd:hd + 1, :], (((1,), (1,)), ((), ())),
                preferred_element_type=jnp.float32))
        cols.extend(vh)
    v = jnp.concatenate([cols[0], cols[1], cols[2], cols[3]], axis=1)
    o_ref[...] = lax.dot_general(
        v, x_ref[...], (((0,), (1,)), ((), ())),
        preferred_element_type=jnp.float32)


def _make_eler(f, f2):
    return pl.pallas_call(
        _eler_body,
        in_specs=[pl.BlockSpec((N, f), lambda: (0, 0)),
                  pl.BlockSpec((f, H * f2), lambda: (0, 0)),
                  pl.BlockSpec((H, f2), lambda: (0, 0)),
                  pl.BlockSpec((H, f2), lambda: (0, 0))],
        out_specs=pl.BlockSpec((4, N), lambda: (0, 0)),
        out_shape=jax.ShapeDtypeStruct((4, N), jnp.float32),
    )


_eler0_call = _make_eler(D, D)
_eler1_call = _make_eler(H * D, H * D)


def _stats_body(nt, o_ref, g_ref, s1_ref, s2_ref, cnt_ref):
    i = pl.program_id(0)
    g = g_ref[0, 0, :]
    m = (lax.broadcasted_iota(jnp.int32, (B, BN), 0)
         == g[None, :]).astype(jnp.float32)

    @pl.when(i == 0)
    def _():
        s1_ref[...] = jnp.zeros_like(s1_ref)
        s2_ref[...] = jnp.zeros_like(s2_ref)
        cnt_ref[...] = jnp.zeros_like(cnt_ref)

    cnt_ref[...] += jnp.broadcast_to(
        jnp.sum(m, axis=1, keepdims=True), (B, D))
    for t in range(nt):
        ot = o_ref[t]
        agg = ot[:, :D] / (ot[:, D:D + 1] + 1e-16)
        s1_ref[:, t * D:(t + 1) * D] += jnp.dot(
            m, agg, preferred_element_type=jnp.float32)
        s2_ref[:, t * D:(t + 1) * D] += jnp.dot(
            m, agg * agg, preferred_element_type=jnp.float32)


def _make_stats(nt):
    return pl.pallas_call(
        functools.partial(_stats_body, nt),
        grid=(GRID,),
        in_specs=[pl.BlockSpec((nt, BN, R), lambda i: (0, i, 0)),
                  pl.BlockSpec((1, 1, BN), lambda i: (i, 0, 0))],
        out_specs=[pl.BlockSpec((B, nt * D), lambda i: (0, 0)),
                   pl.BlockSpec((B, nt * D), lambda i: (0, 0)),
                   pl.BlockSpec((B, D), lambda i: (0, 0))],
        out_shape=[jax.ShapeDtypeStruct((B, nt * D), jnp.float32),
                   jax.ShapeDtypeStruct((B, nt * D), jnp.float32),
                   jax.ShapeDtypeStruct((B, D), jnp.float32)],
    )


_stats2_call = _make_stats(H)
_stats4_call = _make_stats(2 * H)


def _norm_parts(nt, o_ref, g, s1_ref, s2_ref, cnt, w_ref, b_ref, ms_ref):
    mt = (g[:, None] == lax.broadcasted_iota(
        jnp.int32, (BN, B), 1)).astype(jnp.float32)
    mean = s1_ref[...] / cnt
    ms = ms_ref[...]
    var = s2_ref[...] / cnt - mean * mean * ms * (2.0 - ms)
    mean_n = jnp.dot(mt, mean, preferred_element_type=jnp.float32)
    var_n = jnp.dot(mt, var, preferred_element_type=jnp.float32)
    ys = []
    for t in range(nt):
        ot = o_ref[t]
        agg = ot[:, :D] / (ot[:, D:D + 1] + 1e-16)
        sl = slice(t * D, (t + 1) * D)
        yt = (w_ref[sl] * (agg - ms[sl] * mean_n[:, sl])
              * lax.rsqrt(var_n[:, sl] + 1e-5) + b_ref[sl])
        ys.append(jnp.where(yt >= 0.0, yt, 0.01 * yt))
    return ys


def _post0_body(o_ref, g_ref, s1_ref, s2_ref, cnt_ref, w_ref, b_ref, ms_ref,
                w1_ref, y_ref, t1_ref, r0_ref, racc):
    i = pl.program_id(0)
    g = g_ref[0, 0, :]
    cnt = jnp.maximum(cnt_ref[:, 0:1], 1.0)
    ys = _norm_parts(H, o_ref, g, s1_ref, s2_ref, cnt, w_ref, b_ref, ms_ref)
    y = jnp.concatenate(ys, axis=1)
    y_ref[...] = y
    m = (lax.broadcasted_iota(jnp.int32, (B, BN), 0)
         == g[None, :]).astype(jnp.float32)

    @pl.when(i == 0)
    def _():
        racc[...] = jnp.zeros_like(racc)

    racc[...] += jnp.dot(m, 0.5 * (ys[0] + ys[1]),
                         preferred_element_type=jnp.float32)

    @pl.when(i == GRID - 1)
    def _():
        r0_ref[...] = racc[...] / cnt

    h1 = jnp.dot(y, w1_ref[...], preferred_element_type=jnp.float32)
    ones = jnp.ones((BN, 1), jnp.float32)
    zeros = jnp.zeros((BN, R - D - 1), jnp.float32)
    for t in range(2 * H):
        hd, q = t // 2, t % 2
        base = hd * (H * D) + q * D
        t1_ref[t] = jnp.concatenate(
            [h1[:, base:base + D], ones, zeros], axis=1)


_post0_call = pl.pallas_call(
    _post0_body,
    grid=(GRID,),
    in_specs=[pl.BlockSpec((H, BN, R), lambda i: (0, i, 0)),
              pl.BlockSpec((1, 1, BN), lambda i: (i, 0, 0)),
              pl.BlockSpec((B, H * D), lambda i: (0, 0)),
              pl.BlockSpec((B, H * D), lambda i: (0, 0)),
              pl.BlockSpec((B, D), lambda i: (0, 0)),
              pl.BlockSpec((H * D,), lambda i: (0,)),
              pl.BlockSpec((H * D,), lambda i: (0,)),
              pl.BlockSpec((H * D,), lambda i: (0,)),
              pl.BlockSpec((H * D, 2 * H * D), lambda i: (0, 0))],
    out_specs=[pl.BlockSpec((BN, H * D), lambda i: (i, 0)),
               pl.BlockSpec((2 * H, BN, R), lambda i: (0, i, 0)),
               pl.BlockSpec((B, D), lambda i: (0, 0))],
    out_shape=[jax.ShapeDtypeStruct((N, H * D), jnp.float32),
               jax.ShapeDtypeStruct((2 * H, N, R), jnp.float32),
               jax.ShapeDtypeStruct((B, D), jnp.float32)],
    scratch_shapes=[pltpu.VMEM((B, D), jnp.float32)],
)


def _final_body(o_ref, g_ref, s1_ref, s2_ref, cnt_ref, w_ref, b_ref, ms_ref,
                r0_ref, out_ref, racc):
    i = pl.program_id(0)
    g = g_ref[0, 0, :]
    cnt = jnp.maximum(cnt_ref[:, 0:1], 1.0)
    ys = _norm_parts(2 * H, o_ref, g, s1_ref, s2_ref, cnt,
                     w_ref, b_ref, ms_ref)
    feats1 = 0.5 * (jnp.concatenate([ys[0], ys[1]], axis=1)
                    + jnp.concatenate([ys[2], ys[3]], axis=1))
    m = (lax.broadcasted_iota(jnp.int32, (B, BN), 0)
         == g[None, :]).astype(jnp.float32)

    @pl.when(i == 0)
    def _():
        racc[...] = jnp.zeros_like(racc)

    racc[...] += jnp.dot(m, feats1, preferred_element_type=jnp.float32)

    @pl.when(i == GRID - 1)
    def _():
        r0 = r0_ref[...]
        r1 = racc[...] / cnt
        out_ref[:, :D] = jnp.where(r0 >= 0.0, r0, 0.01 * r0)
        out_ref[:, D:] = jnp.where(r1 >= 0.0, r1, 0.01 * r1)


_final_call = pl.pallas_call(
    _final_body,
    grid=(GRID,),
    in_specs=[pl.BlockSpec((2 * H, BN, R), lambda i: (0, i, 0)),
              pl.BlockSpec((1, 1, BN), lambda i: (i, 0, 0)),
              pl.BlockSpec((B, 2 * H * D), lambda i: (0, 0)),
              pl.BlockSpec((B, 2 * H * D), lambda i: (0, 0)),
              pl.BlockSpec((B, D), lambda i: (0, 0)),
              pl.BlockSpec((2 * H * D,), lambda i: (0,)),
              pl.BlockSpec((2 * H * D,), lambda i: (0,)),
              pl.BlockSpec((2 * H * D,), lambda i: (0,)),
              pl.BlockSpec((B, D), lambda i: (0, 0))],
    out_specs=pl.BlockSpec((B, 3 * D), lambda i: (0, 0)),
    out_shape=jax.ShapeDtypeStruct((B, 3 * D), jnp.float32),
    scratch_shapes=[pltpu.VMEM((B, H * D), jnp.float32)],
)


_SC_PARAMS = dict(
    compiler_params=pltpu.CompilerParams(needs_layout_passes=False,
                                         use_tc_tiling_on_sc=False),
)


def _sc_mesh():
    return plsc.VectorSubcoreMesh(core_axis_name="c", subcore_axis_name="s",
                                  num_cores=NC, num_subcores=NS)


def _iota16():
    return lax.broadcasted_iota(jnp.int32, (L,), 0)


def _bucket_body(src_hbm, dst_hbm, qsrc_hbm, qdst_hbm, qcnt_hbm,
                 srcb0, dstb0, srcb1, dstb1, qsrc_v, qdst_v, qcv,
                 sem0, sem1):
    c = lax.axis_index("c")
    s = lax.axis_index("s")
    lo = s * RT
    hi = lo + RT
    nchunks = HE // CS

    def _stage(ch, sb, db, sm):
        off = c * HE + ch * CS
        pltpu.async_copy(src_hbm.at[pl.ds(off, CS)], sb, sm)
        pltpu.async_copy(dst_hbm.at[pl.ds(off, CS)], db, sm)

    def _wait(sb, db, sm):
        pltpu.make_async_copy(src_hbm.at[pl.ds(0, CS)], sb, sm).wait()
        pltpu.make_async_copy(src_hbm.at[pl.ds(0, CS)], db, sm).wait()

    def _scan_buf(sb, db, qp):
        def _scan(i, qp2):
            sv = sb[pl.ds(i * L, L)]
            dv = db[pl.ds(i * L, L)]
            m = (dv >= lo) & (dv < hi)
            plsc.store_compressed(qsrc_v.at[pl.ds(qp2, L)], sv, mask=m)
            plsc.store_compressed(qdst_v.at[pl.ds(qp2, L)], dv - lo, mask=m)
            qn = plsc.all_reduce_population_count(m)
            return jnp.minimum(qp2 + jnp.max(qn), QMAX)
        return lax.fori_loop(0, CS // L, _scan, qp)

    _stage(0, srcb0, dstb0, sem0)

    def _pair(p, qp):
        ch0 = 2 * p
        _stage(ch0 + 1, srcb1, dstb1, sem1)
        _wait(srcb0, dstb0, sem0)
        qp = _scan_buf(srcb0, dstb0, qp)

        @pl.when(ch0 + 2 < nchunks)
        def _():
            _stage(ch0 + 2, srcb0, dstb0, sem0)
        _wait(srcb1, dstb1, sem1)
        return _scan_buf(srcb1, dstb1, qp)
    qp = lax.fori_loop(0, nchunks // 2, _pair, jnp.int32(0))

    padsrc = jnp.full((L,), N, jnp.int32)
    padzero = jnp.zeros((L,), jnp.int32)
    for k in range(CA // L):
        qsrc_v[pl.ds(qp + k * L, L)] = padsrc
        qdst_v[pl.ds(qp + k * L, L)] = padzero
    nch80 = (qp + (QCH - 1)) // QCH
    nchca = (qp + (CA - 1)) // CA
    qcv[...] = jnp.where(_iota16() < 8,
                         jnp.full((L,), nch80, jnp.int32),
                         jnp.full((L,), nchca, jnp.int32))
    pltpu.sync_copy(qsrc_v, qsrc_hbm.at[c, s])
    pltpu.sync_copy(qdst_v, qdst_hbm.at[c, s])
    pltpu.sync_copy(qcv, qcnt_hbm.at[c, s])


def _make_bucket():
    return functools.partial(
        pl.kernel,
        out_type=(jax.ShapeDtypeStruct((NC, NS, CAPQ), jnp.int32),
                  jax.ShapeDtypeStruct((NC, NS, CAPQ), jnp.int32),
                  jax.ShapeDtypeStruct((NC, NS, L), jnp.int32)),
        mesh=_sc_mesh(),
        scratch_types=[
            pltpu.VMEM((CS,), jnp.int32),
            pltpu.VMEM((CS,), jnp.int32),
            pltpu.VMEM((CS,), jnp.int32),
            pltpu.VMEM((CS,), jnp.int32),
            pltpu.VMEM((CAPQ,), jnp.int32),
            pltpu.VMEM((CAPQ,), jnp.int32),
            pltpu.VMEM((L,), jnp.int32),
            pltpu.SemaphoreType.DMA,
            pltpu.SemaphoreType.DMA,
        ],
        **_SC_PARAMS,
    )(_bucket_body)


def _attn_body(qsrc_hbm, qdst_hbm, qcnt_hbm, eler_hbm, ex_hbm,
               el_v, er_v, srcb, dstb, exb, qcv):
    c = lax.axis_index("c")
    s = lax.axis_index("s")
    lo = s * RT
    pltpu.sync_copy(eler_hbm.at[c], el_v)
    pltpu.sync_copy(eler_hbm.at[c + 2, pl.ds(0, N)], er_v)
    iota = _iota16()
    for half in range(NC):
        pltpu.sync_copy(qcnt_hbm.at[half, s], qcv)
        nch = jnp.max(jnp.where(iota >= 8, qcv[...], 0))

        def _chunk(ch, carry):
            off = ch * CA
            pltpu.sync_copy(qsrc_hbm.at[half, s, pl.ds(off, CA)], srcb)
            pltpu.sync_copy(qdst_hbm.at[half, s, pl.ds(off, CA)], dstb)

            def _att(i, carry2):
                sv = srcb[pl.ds(i * L, L)]
                dv = dstb[pl.ds(i * L, L)] + lo
                e = plsc.load_gather(el_v, [sv]) + plsc.load_gather(er_v, [dv])
                e = jnp.where(e >= 0.0, e, 0.2 * e)
                exb[pl.ds(i * L, L)] = jnp.exp(e)
                return carry2
            lax.fori_loop(0, CA // L, _att, 0)
            pltpu.sync_copy(exb, ex_hbm.at[c, half, s, pl.ds(off, CA)])
            return carry
        lax.fori_loop(0, nch, _chunk, 0)


def _make_attn():
    return functools.partial(
        pl.kernel,
        out_type=jax.ShapeDtypeStruct((NC, NC, NS, CAPQ), jnp.float32),
        mesh=_sc_mesh(),
        scratch_types=[
            pltpu.VMEM((N + L,), jnp.float32),
            pltpu.VMEM((N,), jnp.float32),
            pltpu.VMEM((CA,), jnp.int32),
            pltpu.VMEM((CA,), jnp.int32),
            pltpu.VMEM((CA,), jnp.float32),
            pltpu.VMEM((L,), jnp.int32),
        ],
        **_SC_PARAMS,
    )(_attn_body)


def _edge_body(n_passes, t_hbm, qsrc_hbm, qdst_hbm, qcnt_hbm, ex_hbm, o_hbm,
               srcb0, dstb0, exb0, rows0, srcb1, dstb1, exb1, rows1,
               acc, qcv, sem0, sem1):
    c = lax.axis_index("c")
    s = lax.axis_index("s")
    n_tab = NC * n_passes
    zero16 = jnp.zeros((L,), jnp.float32)
    lanes = [_iota16() + j * L for j in range(R // L)]
    bufs = ((srcb0, dstb0, exb0, rows0, sem0),
            (srcb1, dstb1, exb1, rows1, sem1))

    for q in range(n_passes):
        t = c * n_passes + q
        t_off = t * N

        def _zrow(k, carry):
            for j in range(R // L):
                acc[k, pl.ds(j * L, L)] = zero16
            return carry
        lax.fori_loop(0, RTP, _zrow, 0)

        for half in range(NC):
            pltpu.sync_copy(qcnt_hbm.at[half, s], qcv)
            nch = jnp.max(jnp.where(_iota16() < 8, qcv[...], 0))

            def _stage(ch2, b):
                sb, db, eb, rb, sm = bufs[b]
                off = ch2 * QCH
                pltpu.sync_copy(qsrc_hbm.at[half, s, pl.ds(off, QCH)], sb)
                pltpu.sync_copy(qdst_hbm.at[half, s, pl.ds(off, QCH)], db)
                pltpu.sync_copy(ex_hbm.at[c, half, s, pl.ds(off, QCH)], eb)

                def _boff(i, carry2):
                    sv = sb[pl.ds(i * L, L)]
                    sb[pl.ds(i * L, L)] = jnp.minimum(
                        sv + t_off, n_tab * N - 1)
                    return carry2
                lax.fori_loop(0, QCH // L, _boff, 0)
                pltpu.async_copy(t_hbm.at[sb], rb, sm)

            def _consume(ch, b):
                sb, db, eb, rb, sm = bufs[b]

                @pl.when(ch + 1 < nch)
                def _():
                    _stage(ch + 1, 1 - b)
                pltpu.make_async_copy(t_hbm.at[sb], rb, sm).wait()

                def _accum(i, carry2):
                    isp = jnp.full((L,), i, jnp.int32)
                    li = plsc.load_gather(db, [isp])
                    a = plsc.load_gather(eb, [isp])
                    for j in range(R // L):
                        r = rb[i, pl.ds(j * L, L)] * a
                        plsc.addupdate_scatter(acc, [li, lanes[j]], r)
                    return carry2
                lax.fori_loop(0, QCH, _accum, 0)

            @pl.when(nch > 0)
            def _():
                _stage(0, 0)

            def _chunk(ch, carry):
                @pl.when(lax.rem(ch, 2) == 0)
                def _():
                    _consume(ch, 0)

                @pl.when(lax.rem(ch, 2) == 1)
                def _():
                    _consume(ch, 1)
                return carry
            lax.fori_loop(0, nch, _chunk, 0)

        pltpu.sync_copy(acc.at[pl.ds(0, RT)],
                        o_hbm.at[pl.ds(t_off + s * RT, RT)])


def _make_edge(n_passes):
    n_tab = NC * n_passes
    return functools.partial(
        pl.kernel,
        out_type=jax.ShapeDtypeStruct((n_tab * N, R), jnp.float32),
        mesh=_sc_mesh(),
        scratch_types=[
            pltpu.VMEM((QCH,), jnp.int32),
            pltpu.VMEM((QCH,), jnp.int32),
            pltpu.VMEM((QCH,), jnp.float32),
            pltpu.VMEM((QCH, R), jnp.float32),
            pltpu.VMEM((QCH,), jnp.int32),
            pltpu.VMEM((QCH,), jnp.int32),
            pltpu.VMEM((QCH,), jnp.float32),
            pltpu.VMEM((QCH, R), jnp.float32),
            pltpu.VMEM((RTP, R), jnp.float32),
            pltpu.VMEM((L,), jnp.int32),
            pltpu.SemaphoreType.DMA,
            pltpu.SemaphoreType.DMA,
        ],
        **_SC_PARAMS,
    )(functools.partial(_edge_body, n_passes))


_make_bucket = functools.lru_cache(maxsize=None)(_make_bucket)
_make_attn = functools.lru_cache(maxsize=None)(_make_attn)
_make_edge = functools.lru_cache(maxsize=None)(_make_edge)


def kernel(node_feats, edge_index, graph_ids, W0, a_l0, a_r0, W1, a_l1, a_r1,
           g0_w, g0_b, g0_a, g1_w, g1_b, g1_a):
    src = edge_index[0]
    dst = edge_index[1]
    gids3 = graph_ids.reshape(GRID, 1, BN)

    qsrc, qdst, qcnt = _make_bucket()(src, dst)

    t0 = _t0_call(node_feats, W0)
    eler0 = _eler0_call(node_feats, W0, a_l0, a_r0)
    eler0p = jnp.pad(eler0, ((0, 0), (0, L)), constant_values=-1e30)
    ex0 = _make_attn()(qsrc, qdst, qcnt, eler0p)
    o0 = _make_edge(1)(t0.reshape(H * N, R), qsrc, qdst, qcnt, ex0)
    o0 = o0.reshape(H, N, R)

    w0t = jnp.concatenate([g0_w, g0_w])
    b0t = jnp.concatenate([g0_b, g0_b])
    ms0t = jnp.concatenate([g0_a, g0_a])
    s1, s2, cnt = _stats2_call(o0, gids3)
    y, t1, r0 = _post0_call(o0, gids3, s1, s2, cnt, w0t, b0t, ms0t, W1)

    eler1 = _eler1_call(y, W1, a_l1, a_r1)
    eler1p = jnp.pad(eler1, ((0, 0), (0, L)), constant_values=-1e30)
    ex1 = _make_attn()(qsrc, qdst, qcnt, eler1p)
    o1 = _make_edge(2)(t1.reshape(2 * H * N, R), qsrc, qdst, qcnt, ex1)
    o1 = o1.reshape(2 * H, N, R)

    w1t = jnp.concatenate([g1_w, g1_w])
    b1t = jnp.concatenate([g1_b, g1_b])
    ms1t = jnp.concatenate([g1_a, g1_a])
    s1b, s2b, _ = _stats4_call(o1, gids3)
    return _final_call(o1, gids3, s1b, s2b, cnt, w1t, b1t, ms1t, r0)


# speedup vs baseline: 22.6248x; 1.1602x over previous
import functools

import jax
import jax.numpy as jnp
from jax import lax
from jax.experimental import pallas as pl
from jax.experimental.pallas import tpu as pltpu
from jax.experimental.pallas import tpu_sc as plsc

N = 10000
E = 320000
B = 100
D = 128
H = 2
R = 144
L = 16
NC = 2
NS = 16
BN = 1000
GRID = N // BN
EW = E // NS
HE = E // NC
CS = 1600
QCH = 80
CA = 1040
CAPQ = 13520
QMAX = 12080
RT = N // NS
RTP = 640


def _t0_body(x_ref, w_ref, t_ref):
    h = jnp.dot(x_ref[...], w_ref[...], preferred_element_type=jnp.float32)
    ones = jnp.ones((BN, 1), jnp.float32)
    zeros = jnp.zeros((BN, R - D - 1), jnp.float32)
    for hd in range(H):
        t_ref[hd] = jnp.concatenate(
            [h[:, hd * D:(hd + 1) * D], ones, zeros], axis=1)


_t0_call = pl.pallas_call(
    _t0_body,
    grid=(GRID,),
    in_specs=[pl.BlockSpec((BN, D), lambda i: (i, 0)),
              pl.BlockSpec((D, H * D), lambda i: (0, 0))],
    out_specs=pl.BlockSpec((H, BN, R), lambda i: (0, i, 0)),
    out_shape=jax.ShapeDtypeStruct((H, N, R), jnp.float32),
)


def _eler_body(x_ref, w_ref, al_ref, ar_ref, o_ref):
    w = w_ref[...]
    f2 = al_ref.shape[1]
    cols = []
    for a_ref in (al_ref, ar_ref):
        a = a_ref[...]
        vh = []
        for hd in range(H):
            wh = w[:, hd * f2:(hd + 1) * f2]
            vh.append(lax.dot_general(
                wh, a[hd:hd + 1, :], (((1,), (1,)), ((), ())),
                preferred_element_type=jnp.float32))
        cols.extend(vh)
    v = jnp.concatenate([cols[0], cols[1], cols[2], cols[3]], axis=1)
    o_ref[...] = lax.dot_general(
        v, x_ref[...], (((0,), (1,)), ((), ())),
        preferred_element_type=jnp.float32)


def _make_eler(f, f2):
    return pl.pallas_call(
        _eler_body,
        in_specs=[pl.BlockSpec((N, f), lambda: (0, 0)),
                  pl.BlockSpec((f, H * f2), lambda: (0, 0)),
                  pl.BlockSpec((H, f2), lambda: (0, 0)),
                  pl.BlockSpec((H, f2), lambda: (0, 0))],
        out_specs=pl.BlockSpec((4, N), lambda: (0, 0)),
        out_shape=jax.ShapeDtypeStruct((4, N), jnp.float32),
    )


_eler0_call = _make_eler(D, D)
_eler1_call = _make_eler(H * D, H * D)


def _stats_body(nt, o_ref, g_ref, s1_ref, s2_ref, cnt_ref):
    i = pl.program_id(0)
    g = g_ref[0, 0, :]
    m = (lax.broadcasted_iota(jnp.int32, (B, BN), 0)
         == g[None, :]).astype(jnp.float32)

    @pl.when(i == 0)
    def _():
        s1_ref[...] = jnp.zeros_like(s1_ref)
        s2_ref[...] = jnp.zeros_like(s2_ref)
        cnt_ref[...] = jnp.zeros_like(cnt_ref)

    cnt_ref[...] += jnp.broadcast_to(
        jnp.sum(m, axis=1, keepdims=True), (B, D))
    for t in range(nt):
        ot = o_ref[t]
        agg = ot[:, :D] / (ot[:, D:D + 1] + 1e-16)
        s1_ref[:, t * D:(t + 1) * D] += jnp.dot(
            m, agg, preferred_element_type=jnp.float32)
        s2_ref[:, t * D:(t + 1) * D] += jnp.dot(
            m, agg * agg, preferred_element_type=jnp.float32)


def _make_stats(nt):
    return pl.pallas_call(
        functools.partial(_stats_body, nt),
        grid=(GRID,),
        in_specs=[pl.BlockSpec((nt, BN, R), lambda i: (0, i, 0)),
                  pl.BlockSpec((1, 1, BN), lambda i: (i, 0, 0))],
        out_specs=[pl.BlockSpec((B, nt * D), lambda i: (0, 0)),
                   pl.BlockSpec((B, nt * D), lambda i: (0, 0)),
                   pl.BlockSpec((B, D), lambda i: (0, 0))],
        out_shape=[jax.ShapeDtypeStruct((B, nt * D), jnp.float32),
                   jax.ShapeDtypeStruct((B, nt * D), jnp.float32),
                   jax.ShapeDtypeStruct((B, D), jnp.float32)],
    )


_stats2_call = _make_stats(H)
_stats4_call = _make_stats(2 * H)


def _norm_parts(nt, o_ref, g, s1_ref, s2_ref, cnt, w_ref, b_ref, ms_ref):
    mt = (g[:, None] == lax.broadcasted_iota(
        jnp.int32, (BN, B), 1)).astype(jnp.float32)
    mean = s1_ref[...] / cnt
    ms = ms_ref[...]
    var = s2_ref[...] / cnt - mean * mean * ms * (2.0 - ms)
    mean_n = jnp.dot(mt, mean, preferred_element_type=jnp.float32)
    var_n = jnp.dot(mt, var, preferred_element_type=jnp.float32)
    ys = []
    for t in range(nt):
        ot = o_ref[t]
        agg = ot[:, :D] / (ot[:, D:D + 1] + 1e-16)
        sl = slice(t * D, (t + 1) * D)
        yt = (w_ref[sl] * (agg - ms[sl] * mean_n[:, sl])
              * lax.rsqrt(var_n[:, sl] + 1e-5) + b_ref[sl])
        ys.append(jnp.where(yt >= 0.0, yt, 0.01 * yt))
    return ys


def _post0_body(o_ref, g_ref, s1_ref, s2_ref, cnt_ref, w_ref, b_ref, ms_ref,
                w1_ref, y_ref, t1_ref, r0_ref, racc):
    i = pl.program_id(0)
    g = g_ref[0, 0, :]
    cnt = jnp.maximum(cnt_ref[:, 0:1], 1.0)
    ys = _norm_parts(H, o_ref, g, s1_ref, s2_ref, cnt, w_ref, b_ref, ms_ref)
    y = jnp.concatenate(ys, axis=1)
    y_ref[...] = y
    m = (lax.broadcasted_iota(jnp.int32, (B, BN), 0)
         == g[None, :]).astype(jnp.float32)

    @pl.when(i == 0)
    def _():
        racc[...] = jnp.zeros_like(racc)

    racc[...] += jnp.dot(m, 0.5 * (ys[0] + ys[1]),
                         preferred_element_type=jnp.float32)

    @pl.when(i == GRID - 1)
    def _():
        r0_ref[...] = racc[...] / cnt

    h1 = jnp.dot(y, w1_ref[...], preferred_element_type=jnp.float32)
    ones = jnp.ones((BN, 1), jnp.float32)
    zeros = jnp.zeros((BN, R - D - 1), jnp.float32)
    for t in range(2 * H):
        hd, q = t // 2, t % 2
        base = hd * (H * D) + q * D
        t1_ref[t] = jnp.concatenate(
            [h1[:, base:base + D], ones, zeros], axis=1)


_post0_call = pl.pallas_call(
    _post0_body,
    grid=(GRID,),
    in_specs=[pl.BlockSpec((H, BN, R), lambda i: (0, i, 0)),
              pl.BlockSpec((1, 1, BN), lambda i: (i, 0, 0)),
              pl.BlockSpec((B, H * D), lambda i: (0, 0)),
              pl.BlockSpec((B, H * D), lambda i: (0, 0)),
              pl.BlockSpec((B, D), lambda i: (0, 0)),
              pl.BlockSpec((H * D,), lambda i: (0,)),
              pl.BlockSpec((H * D,), lambda i: (0,)),
              pl.BlockSpec((H * D,), lambda i: (0,)),
              pl.BlockSpec((H * D, 2 * H * D), lambda i: (0, 0))],
    out_specs=[pl.BlockSpec((BN, H * D), lambda i: (i, 0)),
               pl.BlockSpec((2 * H, BN, R), lambda i: (0, i, 0)),
               pl.BlockSpec((B, D), lambda i: (0, 0))],
    out_shape=[jax.ShapeDtypeStruct((N, H * D), jnp.float32),
               jax.ShapeDtypeStruct((2 * H, N, R), jnp.float32),
               jax.ShapeDtypeStruct((B, D), jnp.float32)],
    scratch_shapes=[pltpu.VMEM((B, D), jnp.float32)],
)


def _final_body(o_ref, g_ref, s1_ref, s2_ref, cnt_ref, w_ref, b_ref, ms_ref,
                r0_ref, out_ref, racc):
    i = pl.program_id(0)
    g = g_ref[0, 0, :]
    cnt = jnp.maximum(cnt_ref[:, 0:1], 1.0)
    ys = _norm_parts(2 * H, o_ref, g, s1_ref, s2_ref, cnt,
                     w_ref, b_ref, ms_ref)
    feats1 = 0.5 * (jnp.concatenate([ys[0], ys[1]], axis=1)
                    + jnp.concatenate([ys[2], ys[3]], axis=1))
    m = (lax.broadcasted_iota(jnp.int32, (B, BN), 0)
         == g[None, :]).astype(jnp.float32)

    @pl.when(i == 0)
    def _():
        racc[...] = jnp.zeros_like(racc)

    racc[...] += jnp.dot(m, feats1, preferred_element_type=jnp.float32)

    @pl.when(i == GRID - 1)
    def _():
        r0 = r0_ref[...]
        r1 = racc[...] / cnt
        out_ref[:, :D] = jnp.where(r0 >= 0.0, r0, 0.01 * r0)
        out_ref[:, D:] = jnp.where(r1 >= 0.0, r1, 0.01 * r1)


_final_call = pl.pallas_call(
    _final_body,
    grid=(GRID,),
    in_specs=[pl.BlockSpec((2 * H, BN, R), lambda i: (0, i, 0)),
              pl.BlockSpec((1, 1, BN), lambda i: (i, 0, 0)),
              pl.BlockSpec((B, 2 * H * D), lambda i: (0, 0)),
              pl.BlockSpec((B, 2 * H * D), lambda i: (0, 0)),
              pl.BlockSpec((B, D), lambda i: (0, 0)),
              pl.BlockSpec((2 * H * D,), lambda i: (0,)),
              pl.BlockSpec((2 * H * D,), lambda i: (0,)),
              pl.BlockSpec((2 * H * D,), lambda i: (0,)),
              pl.BlockSpec((B, D), lambda i: (0, 0))],
    out_specs=pl.BlockSpec((B, 3 * D), lambda i: (0, 0)),
    out_shape=jax.ShapeDtypeStruct((B, 3 * D), jnp.float32),
    scratch_shapes=[pltpu.VMEM((B, H * D), jnp.float32)],
)


_SC_PARAMS = dict(
    compiler_params=pltpu.CompilerParams(needs_layout_passes=False,
                                         use_tc_tiling_on_sc=False),
)


def _sc_mesh():
    return plsc.VectorSubcoreMesh(core_axis_name="c", subcore_axis_name="s",
                                  num_cores=NC, num_subcores=NS)


def _iota16():
    return lax.broadcasted_iota(jnp.int32, (L,), 0)


def _bucket_body(src_hbm, dst_hbm, qsrc_hbm, qdst_hbm, qcnt_hbm, qsrc4_hbm,
                 srcb0, dstb0, srcb1, dstb1, qsrc_v, qdst_v, qcv, qtmp,
                 sem0, sem1):
    c = lax.axis_index("c")
    s = lax.axis_index("s")
    lo = s * RT
    hi = lo + RT
    nchunks = HE // CS

    def _stage(ch, sb, db, sm):
        off = c * HE + ch * CS
        pltpu.async_copy(src_hbm.at[pl.ds(off, CS)], sb, sm)
        pltpu.async_copy(dst_hbm.at[pl.ds(off, CS)], db, sm)

    def _wait(sb, db, sm):
        pltpu.make_async_copy(src_hbm.at[pl.ds(0, CS)], sb, sm).wait()
        pltpu.make_async_copy(src_hbm.at[pl.ds(0, CS)], db, sm).wait()

    def _scan_buf(sb, db, qp):
        def _scan(i, qp2):
            sv = sb[pl.ds(i * L, L)]
            dv = db[pl.ds(i * L, L)]
            m = (dv >= lo) & (dv < hi)
            plsc.store_compressed(qsrc_v.at[pl.ds(qp2, L)], sv, mask=m)
            plsc.store_compressed(qdst_v.at[pl.ds(qp2, L)], dv - lo, mask=m)
            qn = plsc.all_reduce_population_count(m)
            return jnp.minimum(qp2 + jnp.max(qn), QMAX)
        return lax.fori_loop(0, CS // L, _scan, qp)

    _stage(0, srcb0, dstb0, sem0)

    def _pair(p, qp):
        ch0 = 2 * p
        _stage(ch0 + 1, srcb1, dstb1, sem1)
        _wait(srcb0, dstb0, sem0)
        qp = _scan_buf(srcb0, dstb0, qp)

        @pl.when(ch0 + 2 < nchunks)
        def _():
            _stage(ch0 + 2, srcb0, dstb0, sem0)
        _wait(srcb1, dstb1, sem1)
        return _scan_buf(srcb1, dstb1, qp)
    qp = lax.fori_loop(0, nchunks // 2, _pair, jnp.int32(0))

    padsrc = jnp.full((L,), N, jnp.int32)
    padzero = jnp.zeros((L,), jnp.int32)
    for k in range(CA // L):
        qsrc_v[pl.ds(qp + k * L, L)] = padsrc
        qdst_v[pl.ds(qp + k * L, L)] = padzero
    nch80 = (qp + (QCH - 1)) // QCH
    nchca = (qp + (CA - 1)) // CA
    qcv[...] = jnp.where(_iota16() < 8,
                         jnp.full((L,), nch80, jnp.int32),
                         jnp.full((L,), nchca, jnp.int32))
    pltpu.sync_copy(qsrc_v, qsrc_hbm.at[c, s])
    pltpu.sync_copy(qdst_v, qdst_hbm.at[c, s])
    pltpu.sync_copy(qcv, qcnt_hbm.at[c, s])
    for t in range(2 * H):
        def _qoff(k, carry):
            sv = qsrc_v[pl.ds(k * L, L)]
            qtmp[pl.ds(k * L, L)] = jnp.minimum(sv + t * N, (t + 1) * N - 1)
            return carry
        lax.fori_loop(0, CAPQ // L, _qoff, 0)
        pltpu.sync_copy(qtmp, qsrc4_hbm.at[t, c, s])


def _make_bucket():
    return functools.partial(
        pl.kernel,
        out_type=(jax.ShapeDtypeStruct((NC, NS, CAPQ), jnp.int32),
                  jax.ShapeDtypeStruct((NC, NS, CAPQ), jnp.int32),
                  jax.ShapeDtypeStruct((NC, NS, L), jnp.int32),
                  jax.ShapeDtypeStruct((2 * H, NC, NS, CAPQ), jnp.int32)),
        mesh=_sc_mesh(),
        scratch_types=[
            pltpu.VMEM((CS,), jnp.int32),
            pltpu.VMEM((CS,), jnp.int32),
            pltpu.VMEM((CS,), jnp.int32),
            pltpu.VMEM((CS,), jnp.int32),
            pltpu.VMEM((CAPQ,), jnp.int32),
            pltpu.VMEM((CAPQ,), jnp.int32),
            pltpu.VMEM((L,), jnp.int32),
            pltpu.VMEM((CAPQ,), jnp.int32),
            pltpu.SemaphoreType.DMA,
            pltpu.SemaphoreType.DMA,
        ],
        **_SC_PARAMS,
    )(_bucket_body)


def _attn_body(qsrc_hbm, qdst_hbm, qcnt_hbm, eler_hbm, ex_hbm,
               el_v, er_v, srcb, dstb, exb, qcv):
    c = lax.axis_index("c")
    s = lax.axis_index("s")
    lo = s * RT
    pltpu.sync_copy(eler_hbm.at[c], el_v)
    pltpu.sync_copy(eler_hbm.at[c + 2, pl.ds(0, N)], er_v)
    iota = _iota16()
    for half in range(NC):
        pltpu.sync_copy(qcnt_hbm.at[half, s], qcv)
        nch = jnp.max(jnp.where(iota >= 8, qcv[...], 0))

        def _chunk(ch, carry):
            off = ch * CA
            pltpu.sync_copy(qsrc_hbm.at[half, s, pl.ds(off, CA)], srcb)
            pltpu.sync_copy(qdst_hbm.at[half, s, pl.ds(off, CA)], dstb)

            def _att(i, carry2):
                sv = srcb[pl.ds(i * L, L)]
                dv = dstb[pl.ds(i * L, L)] + lo
                e = plsc.load_gather(el_v, [sv]) + plsc.load_gather(er_v, [dv])
                e = jnp.where(e >= 0.0, e, 0.2 * e)
                exb[pl.ds(i * L, L)] = jnp.exp(e)
                return carry2
            lax.fori_loop(0, CA // L, _att, 0)
            pltpu.sync_copy(exb, ex_hbm.at[c, half, s, pl.ds(off, CA)])
            return carry
        lax.fori_loop(0, nch, _chunk, 0)


def _make_attn():
    return functools.partial(
        pl.kernel,
        out_type=jax.ShapeDtypeStruct((NC, NC, NS, CAPQ), jnp.float32),
        mesh=_sc_mesh(),
        scratch_types=[
            pltpu.VMEM((N + L,), jnp.float32),
            pltpu.VMEM((N,), jnp.float32),
            pltpu.VMEM((CA,), jnp.int32),
            pltpu.VMEM((CA,), jnp.int32),
            pltpu.VMEM((CA,), jnp.float32),
            pltpu.VMEM((L,), jnp.int32),
        ],
        **_SC_PARAMS,
    )(_attn_body)


def _edge_body(n_passes, t_hbm, qsrc4_hbm, qdst_hbm, qcnt_hbm, ex_hbm, o_hbm,
               srcb0, dstb0, exb0, rows0, srcb1, dstb1, exb1, rows1,
               acc, qcv, semi0, semi1, semg0, semg1):
    c = lax.axis_index("c")
    s = lax.axis_index("s")
    zero16 = jnp.zeros((L,), jnp.float32)
    lanes = [_iota16() + j * L for j in range(R // L)]
    bufs = ((srcb0, dstb0, exb0, rows0, semi0, semg0),
            (srcb1, dstb1, exb1, rows1, semi1, semg1))

    for q in range(n_passes):
        t = c * n_passes + q
        t_off = t * N

        def _zrow(k, carry):
            for j in range(R // L):
                acc[k, pl.ds(j * L, L)] = zero16
            return carry
        lax.fori_loop(0, RTP, _zrow, 0)

        for half in range(NC):
            pltpu.sync_copy(qcnt_hbm.at[half, s], qcv)
            nch = jnp.max(jnp.where(_iota16() < 8, qcv[...], 0))

            def _stage_idx(ch2, b):
                sb, db, eb, rb, smi, smg = bufs[b]
                off = ch2 * QCH
                pltpu.async_copy(
                    qsrc4_hbm.at[t, half, s, pl.ds(off, QCH)], sb, smi)
                pltpu.async_copy(
                    qdst_hbm.at[half, s, pl.ds(off, QCH)], db, smi)
                pltpu.async_copy(
                    ex_hbm.at[c, half, s, pl.ds(off, QCH)], eb, smi)

            def _launch(b):
                sb, db, eb, rb, smi, smg = bufs[b]
                pltpu.make_async_copy(
                    qsrc4_hbm.at[0, 0, 0, pl.ds(0, QCH)], sb, smi).wait()
                pltpu.make_async_copy(
                    qdst_hbm.at[0, 0, pl.ds(0, QCH)], db, smi).wait()
                pltpu.make_async_copy(
                    ex_hbm.at[0, 0, 0, pl.ds(0, QCH)], eb, smi).wait()
                pltpu.async_copy(t_hbm.at[sb], rb, smg)

            def _consume(ch, b):
                sb, db, eb, rb, smi, smg = bufs[b]

                @pl.when(ch + 1 < nch)
                def _():
                    _launch(1 - b)
                pltpu.make_async_copy(t_hbm.at[sb], rb, smg).wait()

                def _accum(i, carry2):
                    isp = jnp.full((L,), i, jnp.int32)
                    li = plsc.load_gather(db, [isp])
                    a = plsc.load_gather(eb, [isp])
                    for j in range(R // L):
                        r = rb[i, pl.ds(j * L, L)] * a
                        plsc.addupdate_scatter(acc, [li, lanes[j]], r)
                    return carry2
                lax.fori_loop(0, QCH, _accum, 0)

                @pl.when(ch + 2 < nch)
                def _():
                    _stage_idx(ch + 2, b)

            @pl.when(nch > 0)
            def _():
                _stage_idx(0, 0)

            @pl.when(nch > 1)
            def _():
                _stage_idx(1, 1)

            @pl.when(nch > 0)
            def _():
                _launch(0)

            def _chunk(ch, carry):
                @pl.when(lax.rem(ch, 2) == 0)
                def _():
                    _consume(ch, 0)

                @pl.when(lax.rem(ch, 2) == 1)
                def _():
                    _consume(ch, 1)
                return carry
            lax.fori_loop(0, nch, _chunk, 0)

        pltpu.sync_copy(acc.at[pl.ds(0, RT)],
                        o_hbm.at[pl.ds(t_off + s * RT, RT)])


def _make_edge(n_passes):
    n_tab = NC * n_passes
    return functools.partial(
        pl.kernel,
        out_type=jax.ShapeDtypeStruct((n_tab * N, R), jnp.float32),
        mesh=_sc_mesh(),
        scratch_types=[
            pltpu.VMEM((QCH,), jnp.int32),
            pltpu.VMEM((QCH,), jnp.int32),
            pltpu.VMEM((QCH,), jnp.float32),
            pltpu.VMEM((QCH, R), jnp.float32),
            pltpu.VMEM((QCH,), jnp.int32),
            pltpu.VMEM((QCH,), jnp.int32),
            pltpu.VMEM((QCH,), jnp.float32),
            pltpu.VMEM((QCH, R), jnp.float32),
            pltpu.VMEM((RTP, R), jnp.float32),
            pltpu.VMEM((L,), jnp.int32),
            pltpu.SemaphoreType.DMA,
            pltpu.SemaphoreType.DMA,
            pltpu.SemaphoreType.DMA,
            pltpu.SemaphoreType.DMA,
        ],
        **_SC_PARAMS,
    )(functools.partial(_edge_body, n_passes))


_make_bucket = functools.lru_cache(maxsize=None)(_make_bucket)
_make_attn = functools.lru_cache(maxsize=None)(_make_attn)
_make_edge = functools.lru_cache(maxsize=None)(_make_edge)


def kernel(node_feats, edge_index, graph_ids, W0, a_l0, a_r0, W1, a_l1, a_r1,
           g0_w, g0_b, g0_a, g1_w, g1_b, g1_a):
    src = edge_index[0]
    dst = edge_index[1]
    gids3 = graph_ids.reshape(GRID, 1, BN)

    qsrc, qdst, qcnt, qsrc4 = _make_bucket()(src, dst)

    t0 = _t0_call(node_feats, W0)
    eler0 = _eler0_call(node_feats, W0, a_l0, a_r0)
    eler0p = jnp.pad(eler0, ((0, 0), (0, L)), constant_values=-1e30)
    ex0 = _make_attn()(qsrc, qdst, qcnt, eler0p)
    o0 = _make_edge(1)(t0.reshape(H * N, R), qsrc4, qdst, qcnt, ex0)
    o0 = o0.reshape(H, N, R)

    w0t = jnp.concatenate([g0_w, g0_w])
    b0t = jnp.concatenate([g0_b, g0_b])
    ms0t = jnp.concatenate([g0_a, g0_a])
    s1, s2, cnt = _stats2_call(o0, gids3)
    y, t1, r0 = _post0_call(o0, gids3, s1, s2, cnt, w0t, b0t, ms0t, W1)

    eler1 = _eler1_call(y, W1, a_l1, a_r1)
    eler1p = jnp.pad(eler1, ((0, 0), (0, L)), constant_values=-1e30)
    ex1 = _make_attn()(qsrc, qdst, qcnt, eler1p)
    o1 = _make_edge(2)(t1.reshape(2 * H * N, R), qsrc4, qdst, qcnt, ex1)
    o1 = o1.reshape(2 * H, N, R)

    w1t = jnp.concatenate([g1_w, g1_w])
    b1t = jnp.concatenate([g1_b, g1_b])
    ms1t = jnp.concatenate([g1_a, g1_a])
    s1b, s2b, _ = _stats4_call(o1, gids3)
    return _final_call(o1, gids3, s1b, s2b, cnt, w1t, b1t, ms1t, r0)
